# baseline (speedup 1.0000x reference)
import functools
import numpy as np
import jax
import jax.numpy as jnp
from jax import lax
from jax.experimental import pallas as pl
from jax.experimental.pallas import tpu as pltpu
from jax.experimental.pallas import tpu_sc as plsc

B = 16384
N_FEAT = 26
EMB = 64
QUANTILES = np.array([-1.2816, -0.8416, -0.5244, -0.2533, 0.0,
                      0.2533, 0.5244, 0.8416, 1.2816], dtype=np.float32)
NP0, NB0 = 16, 20
NP1, NB1 = 32, 50
C0 = NP0 * (NB0 + 1)
C1 = NP1 * (NB1 + 1)
C = C0 + C1
NPROJ = NP0 + NP1

NEG = np.float32(-3.0e38)
POS = np.float32(3.0e38)


def _grid_pts(nb):
    res = 2.0 / nb
    return (np.linspace(-1.0, 1.0, nb + 1)[:-1] + 0.5 * res).astype(np.float32)


_TH = np.concatenate([
    np.repeat(np.concatenate([[NEG], _grid_pts(NB0)]).astype(np.float32), NP0),
    np.repeat(np.concatenate([[NEG], _grid_pts(NB1)]).astype(np.float32), NP1),
])

_THMAP = np.full((NB1, NPROJ), POS, dtype=np.float32)
_THMAP[:NB0, :NP0] = _grid_pts(NB0)[:, None]
_THMAP[:NB1, NP0:] = _grid_pts(NB1)[:, None]
_OFFS = np.concatenate([
    (NB0 + 1) * np.arange(NP0), C0 + (NB1 + 1) * np.arange(NP1),
]).astype(np.int32)

BB = 2048
BSC = 4096
BBM = 2048
NW = 32
RPW = BSC // NW
NIR = RPW * 48 // 128
NSC = RPW // 8


def _dense_map(x, p_ref):
    b = jnp.zeros_like(x)
    for q in QUANTILES:
        b += (x > q).astype(jnp.float32)
    xq = b / np.float32(10.0) - np.float32(0.5)
    n = jnp.sqrt(jnp.sum(xq * xq, axis=1, keepdims=True))
    xn = xq / jnp.maximum(n, np.float32(1e-12))
    return jnp.dot(xn, p_ref[...], preferred_element_type=jnp.float32)


def _thermo_body(x_ref, p_ref, th_ref, w0_ref, w1_ref, o_ref, dw_ref):
    @pl.when(pl.program_id(0) == 0)
    def _prep():
        w0 = w0_ref[...]
        w1 = w1_ref[...]
        dw0 = w0 - jnp.concatenate(
            [jnp.zeros((NP0, EMB), jnp.float32), w0[:C0 - NP0]], axis=0)
        dw1 = w1 - jnp.concatenate(
            [jnp.zeros((NP1, EMB), jnp.float32), w1[:C1 - NP1]], axis=0)
        dw = jnp.concatenate([dw0, dw1], axis=0)
        dwh = dw.astype(jnp.bfloat16)
        dwl = (dw - dwh.astype(jnp.float32)).astype(jnp.bfloat16)
        dw_ref[...] = jnp.concatenate([dwh, dwl], axis=1)

    z = _dense_map(x_ref[...], p_ref)
    z0 = z[:, :NP0]
    z1 = z[:, NP0:]
    zt = jnp.concatenate([z0] * (NB0 + 1) + [z1] * (NB1 + 1), axis=1)
    s = (zt > th_ref[...]).astype(jnp.bfloat16)
    acc2 = jnp.dot(s, dw_ref[...], preferred_element_type=jnp.float32)
    o_ref[...] = acc2[:, :EMB] + acc2[:, EMB:]


def _mapper_body(x_ref, p_ref, thm_ref, off_ref, o_ref):
    z = _dense_map(x_ref[...], p_ref)
    thm = thm_ref[...]
    cnt = jnp.zeros((BBM, NPROJ), jnp.float32)
    for j in range(NB1):
        cnt += (z > thm[j:j + 1, :]).astype(jnp.float32)
    o_ref[...] = cnt.astype(jnp.int32) + off_ref[...]


def _make_bag():
    mesh = plsc.VectorSubcoreMesh(core_axis_name="c", subcore_axis_name="s")

    @functools.partial(
        pl.kernel, mesh=mesh,
        compiler_params=pltpu.CompilerParams(use_tc_tiling_on_sc=False),
        out_type=jax.ShapeDtypeStruct((BSC, EMB), jnp.float32),
        scratch_types=[
            pltpu.VMEM((NIR, 128), jnp.int32),
            pltpu.VMEM((2, 384, EMB), jnp.float32),
            pltpu.VMEM((RPW, EMB), jnp.float32),
            pltpu.SemaphoreType.DMA,
            pltpu.SemaphoreType.DMA,
        ],
    )
    def bag(table_hbm, idx_hbm, out_hbm, idx_v, rows_v, out_v, sem0, sem1):
        wid = lax.axis_index("s") * 2 + lax.axis_index("c")
        pltpu.sync_copy(idx_hbm.at[pl.ds(wid * NIR, NIR)], idx_v)

        def _fire(sc, slot, sem):
            for j in range(3):
                pltpu.async_copy(
                    table_hbm.at[idx_v.at[sc * 3 + j]],
                    rows_v.at[slot, pl.ds(j * 128, 128)], sem)

        def _drain(sc, slot, sem):
            for j in range(3):
                pltpu.make_async_copy(
                    table_hbm.at[idx_v.at[sc * 3 + j]],
                    rows_v.at[slot, pl.ds(j * 128, 128)], sem).wait()

        def _accum(sc, slot):
            cur = rows_v.at[slot]
            for rr in range(8):
                r = sc * 8 + rr
                for d in range(EMB // 16):
                    acc = cur[rr * 48, pl.ds(d * 16, 16)]
                    for k in range(1, 48):
                        acc = acc + cur[rr * 48 + k, pl.ds(d * 16, 16)]
                    out_v[r, pl.ds(d * 16, 16)] = acc

        _fire(0, 0, sem0)

        def pair_step(i, _):
            sc = i * 2
            _fire(sc + 1, 1, sem1)
            _drain(sc, 0, sem0)
            _accum(sc, 0)

            @pl.when(i + 1 < NSC // 2)
            def _next():
                _fire(sc + 2, 0, sem0)
            _drain(sc + 1, 1, sem1)
            _accum(sc + 1, 1)
            return 0

        lax.fori_loop(0, NSC // 2, pair_step, 0)
        pltpu.sync_copy(out_v, out_hbm.at[pl.ds(wid * RPW, RPW)])

    return bag


def kernel(f00, f01, f02, f03, f04, f05, f06, f07, f08, f09, f10, f11,
           f12, f13, f14, f15, f16, f17, f18, f19, f20, f21, f22, f23,
           f24, f25, proj0, proj1, w0, w1):
    feats = [f00, f01, f02, f03, f04, f05, f06, f07, f08, f09, f10, f11,
             f12, f13, f14, f15, f16, f17, f18, f19, f20, f21, f22, f23,
             f24, f25]
    x = jnp.concatenate(feats, axis=1)
    p = jnp.concatenate([proj0, proj1], axis=1)
    w0r = w0.reshape(NP0, NB0 + 1, EMB).transpose(1, 0, 2).reshape(C0, EMB)
    w1r = w1.reshape(NP1, NB1 + 1, EMB).transpose(1, 0, 2).reshape(C1, EMB)
    th = jnp.asarray(_TH)[None, :]
    table = jnp.concatenate([w0, w1], axis=0)

    idx = pl.pallas_call(
        _mapper_body,
        grid=(BSC // BBM,),
        in_specs=[
            pl.BlockSpec((BBM, N_FEAT), lambda i: (i, 0)),
            pl.BlockSpec((N_FEAT, NPROJ), lambda i: (0, 0)),
            pl.BlockSpec((NB1, NPROJ), lambda i: (0, 0)),
            pl.BlockSpec((1, NPROJ), lambda i: (0, 0)),
        ],
        out_specs=pl.BlockSpec((BBM, NPROJ), lambda i: (i, 0)),
        out_shape=jax.ShapeDtypeStruct((BSC, NPROJ), jnp.int32),
    )(x[:BSC], p, jnp.asarray(_THMAP), jnp.asarray(_OFFS)[None, :])
    idx128 = idx.reshape(BSC * 48 // 128, 128)

    out_sc = _make_bag()(table, idx128)

    out_tc = pl.pallas_call(
        _thermo_body,
        grid=((B - BSC) // BB,),
        in_specs=[
            pl.BlockSpec((BB, N_FEAT), lambda i: (i, 0)),
            pl.BlockSpec((N_FEAT, NPROJ), lambda i: (0, 0)),
            pl.BlockSpec((1, C), lambda i: (0, 0)),
            pl.BlockSpec((C0, EMB), lambda i: (0, 0)),
            pl.BlockSpec((C1, EMB), lambda i: (0, 0)),
        ],
        out_specs=pl.BlockSpec((BB, EMB), lambda i: (i, 0)),
        out_shape=jax.ShapeDtypeStruct((B - BSC, EMB), jnp.float32),
        scratch_shapes=[pltpu.VMEM((C, 2 * EMB), jnp.bfloat16)],
    )(x[BSC:], p, th, w0r, w1r)
    return jnp.concatenate([out_sc, out_tc], axis=0)


# speedup vs baseline: 2.3700x; 2.3700x over previous
import numpy as np
import jax
import jax.numpy as jnp
from jax.experimental import pallas as pl
from jax.experimental.pallas import tpu as pltpu

B = 16384
N_FEAT = 26
EMB = 64
QUANTILES = np.array([-1.2816, -0.8416, -0.5244, -0.2533, 0.0,
                      0.2533, 0.5244, 0.8416, 1.2816], dtype=np.float32)
NP0, NB0 = 16, 20
NP1, NB1 = 32, 50
C0 = NP0 * (NB0 + 1)
C1 = NP1 * (NB1 + 1)
C = C0 + C1

NEG = np.float32(-3.0e38)


def _grid_pts(nb):
    res = 2.0 / nb
    return (np.linspace(-1.0, 1.0, nb + 1)[:-1] + 0.5 * res).astype(np.float32)


_TH = np.concatenate([
    np.repeat(np.concatenate([[NEG], _grid_pts(NB0)]).astype(np.float32), NP0),
    np.repeat(np.concatenate([[NEG], _grid_pts(NB1)]).astype(np.float32), NP1),
])

BB = 2048


def _body(x_ref, p_ref, th_ref, w0_ref, w1_ref, o_ref, dw_ref):
    @pl.when(pl.program_id(0) == 0)
    def _prep():
        w0 = w0_ref[...]
        w1 = w1_ref[...]
        dw0 = w0 - jnp.concatenate(
            [jnp.zeros((NP0, EMB), jnp.float32), w0[:C0 - NP0]], axis=0)
        dw1 = w1 - jnp.concatenate(
            [jnp.zeros((NP1, EMB), jnp.float32), w1[:C1 - NP1]], axis=0)
        dw = jnp.concatenate([dw0, dw1], axis=0)
        dwh = dw.astype(jnp.bfloat16)
        dwl = (dw - dwh.astype(jnp.float32)).astype(jnp.bfloat16)
        dw_ref[...] = jnp.concatenate([dwh, dwl], axis=1)

    x = x_ref[...]
    b = jnp.zeros_like(x)
    for q in QUANTILES:
        b += (x > q).astype(jnp.float32)
    xq = b / np.float32(10.0) - np.float32(0.5)
    n = jnp.sqrt(jnp.sum(xq * xq, axis=1, keepdims=True))
    xn = xq / jnp.maximum(n, np.float32(1e-12))
    z = jnp.dot(xn, p_ref[...], preferred_element_type=jnp.float32)
    z0 = z[:, :NP0]
    z1 = z[:, NP0:]
    zt = jnp.concatenate([z0] * (NB0 + 1) + [z1] * (NB1 + 1), axis=1)
    s = (zt > th_ref[...]).astype(jnp.bfloat16)
    acc2 = jnp.dot(s, dw_ref[...], preferred_element_type=jnp.float32)
    o_ref[...] = acc2[:, :EMB] + acc2[:, EMB:]


def kernel(f00, f01, f02, f03, f04, f05, f06, f07, f08, f09, f10, f11,
           f12, f13, f14, f15, f16, f17, f18, f19, f20, f21, f22, f23,
           f24, f25, proj0, proj1, w0, w1):
    feats = [f00, f01, f02, f03, f04, f05, f06, f07, f08, f09, f10, f11,
             f12, f13, f14, f15, f16, f17, f18, f19, f20, f21, f22, f23,
             f24, f25]
    x = jnp.concatenate(feats, axis=1)
    p = jnp.concatenate([proj0, proj1], axis=1)
    w0r = w0.reshape(NP0, NB0 + 1, EMB).transpose(1, 0, 2).reshape(C0, EMB)
    w1r = w1.reshape(NP1, NB1 + 1, EMB).transpose(1, 0, 2).reshape(C1, EMB)
    th = jnp.asarray(_TH)[None, :]

    out = pl.pallas_call(
        _body,
        grid=(B // BB,),
        in_specs=[
            pl.BlockSpec((BB, N_FEAT), lambda i: (i, 0)),
            pl.BlockSpec((N_FEAT, NP0 + NP1), lambda i: (0, 0)),
            pl.BlockSpec((1, C), lambda i: (0, 0)),
            pl.BlockSpec((C0, EMB), lambda i: (0, 0)),
            pl.BlockSpec((C1, EMB), lambda i: (0, 0)),
        ],
        out_specs=pl.BlockSpec((BB, EMB), lambda i: (i, 0)),
        out_shape=jax.ShapeDtypeStruct((B, EMB), jnp.float32),
        scratch_shapes=[pltpu.VMEM((C, 2 * EMB), jnp.bfloat16)],
    )(x, p, th, w0r, w1r)
    return out
